# baseline (speedup 1.0000x reference)
import functools

import jax
import jax.numpy as jnp
from jax import lax
from jax.experimental import pallas as pl
from jax.experimental.pallas import tpu as pltpu
from jax.experimental.pallas import tpu_sc as plsc

E = 320000
N = 10000
HID = 128
NRAD = 16
CHUNK = 128
NCH = E // CHUNK
NC = 2
NS = 16
NW = NC * NS
NDRAIN = (N + CHUNK - 1) // CHUNK
TAIL = N - (NDRAIN - 1) * CHUNK


def _sc_gather(m, rb, sb8, ikj2d, iji2d):
    mesh = plsc.VectorSubcoreMesh(core_axis_name="c", subcore_axis_name="s")

    @functools.partial(
        pl.kernel,
        out_type=(
            jax.ShapeDtypeStruct((E, HID), jnp.float32),
            jax.ShapeDtypeStruct((E, NRAD), jnp.float32),
            jax.ShapeDtypeStruct((E, 8), jnp.float32),
        ),
        mesh=mesh,
        scratch_types=[
            pltpu.VMEM((CHUNK,), jnp.int32),
            pltpu.VMEM((CHUNK,), jnp.int32),
            pltpu.VMEM((CHUNK, HID), jnp.float32),
            pltpu.VMEM((CHUNK, NRAD), jnp.float32),
            pltpu.VMEM((CHUNK, 8), jnp.float32),
            pltpu.SemaphoreType.DMA,
            pltpu.SemaphoreType.DMA,
            pltpu.SemaphoreType.DMA,
        ],
        compiler_params=pltpu.CompilerParams(use_tc_tiling_on_sc=False),
    )
    def k(m_hbm, rb_hbm, sb8_hbm, ikj_hbm, iji_hbm,
          mkj_out, rbkj_out, sbji_out,
          ikj_v, iji_v, mbuf, rbbuf, sbbuf, sem0, sem1, sem2):
        wid = lax.axis_index("s") * NC + lax.axis_index("c")

        def body(j, carry):
            chunk = j * NW + wid

            @pl.when(chunk < NCH)
            def _():
                pltpu.sync_copy(ikj_hbm.at[chunk], ikj_v)
                pltpu.sync_copy(iji_hbm.at[chunk], iji_v)
                c0 = pltpu.async_copy(m_hbm.at[ikj_v], mbuf, sem0)
                c1 = pltpu.async_copy(rb_hbm.at[ikj_v], rbbuf, sem1)
                c2 = pltpu.async_copy(sb8_hbm.at[iji_v], sbbuf, sem2)
                c0.wait()
                c1.wait()
                c2.wait()
                base = pl.multiple_of(chunk * CHUNK, CHUNK)
                pltpu.sync_copy(mbuf, mkj_out.at[pl.ds(base, CHUNK)])
                pltpu.sync_copy(rbbuf, rbkj_out.at[pl.ds(base, CHUNK)])
                pltpu.sync_copy(sbbuf, sbji_out.at[pl.ds(base, CHUNK)])

            return carry

        lax.fori_loop(0, (NCH + NW - 1) // NW, body, 0)

    return k(m, rb, sb8, ikj2d, iji2d)


def _sc_scatter(mf, dst2d):
    mesh = plsc.VectorSubcoreMesh(core_axis_name="c", subcore_axis_name="s")

    @functools.partial(
        pl.kernel,
        out_type=jax.ShapeDtypeStruct((NC, N, HID), jnp.float32),
        mesh=mesh,
        scratch_types=[
            pltpu.VMEM((CHUNK,), jnp.int32),
            pltpu.VMEM((CHUNK, HID), jnp.float32),
            pltpu.VMEM_SHARED((N, HID), jnp.float32),
        ],
    )
    def k(mf_hbm, dst_hbm, out_hbm, idx_v, buf, agg_sh):
        c = lax.axis_index("c")
        s = lax.axis_index("s")

        def zrow(i, carry):
            r = i // (HID // 16)
            q = i % (HID // 16)
            buf[r, pl.ds(q * 16, 16)] = jnp.zeros((16,), jnp.float32)
            return carry

        lax.fori_loop(0, CHUNK * (HID // 16), zrow, 0)
        for j in range((NDRAIN + NS - 1) // NS):
            t = j * NS + s

            @pl.when(t < NDRAIN - 1)
            def _():
                pltpu.sync_copy(
                    buf, agg_sh.at[pl.ds(pl.multiple_of(t * CHUNK, CHUNK),
                                         CHUNK)])

            @pl.when(t == NDRAIN - 1)
            def _():
                pltpu.sync_copy(
                    buf.at[pl.ds(0, TAIL)],
                    agg_sh.at[pl.ds((NDRAIN - 1) * CHUNK, TAIL)])

        plsc.subcore_barrier()

        def body(j, carry):
            t = j * NS + s

            @pl.when(t < NCH // NC)
            def _():
                chunk = t * NC + c
                pltpu.sync_copy(dst_hbm.at[chunk], idx_v)
                pltpu.sync_copy(
                    mf_hbm.at[pl.ds(pl.multiple_of(chunk * CHUNK, CHUNK),
                                    CHUNK)], buf)
                pltpu.sync_copy(buf, agg_sh.at[idx_v], add=True)

            return carry

        lax.fori_loop(0, (NCH // NC + NS - 1) // NS, body, 0)
        plsc.subcore_barrier()

        for j in range((NDRAIN + NS - 1) // NS):
            t = j * NS + s

            @pl.when(t < NDRAIN - 1)
            def _():
                base = pl.multiple_of(t * CHUNK, CHUNK)
                pltpu.sync_copy(agg_sh.at[pl.ds(base, CHUNK)],
                                out_hbm.at[c, pl.ds(base, CHUNK)])

            @pl.when(t == NDRAIN - 1)
            def _():
                pltpu.sync_copy(
                    agg_sh.at[pl.ds((NDRAIN - 1) * CHUNK, TAIL)],
                    out_hbm.at[c, pl.ds((NDRAIN - 1) * CHUNK, TAIL)])

    return k(mf, dst2d)


def _tc_edges(m, mkj, rbkj, sbji, W_r1, b_r1, W_r2, b_r2, w8,
              W_o1, b_o1, W_o2, b_o2, W_o3, b_o3):
    BLK = 512

    def body(m_ref, mkj_ref, rb_ref, sb_ref, wr1, br1, wr2, br2, w8r,
             wo1, bo1, wo2, bo2, wo3, bo3, mf_ref, mout_ref):
        silu = jax.nn.silu
        t = silu(jnp.dot(rb_ref[...], wr1[...],
                         preferred_element_type=jnp.float32) + br1[...])
        rf = jnp.dot(t, wr2[...], preferred_element_type=jnp.float32) + br2[...]
        aw = jax.nn.sigmoid(
            jnp.sum(sb_ref[...] * w8r[...], axis=1, keepdims=True))
        mf = mkj_ref[...] * rf * aw
        mf_ref[...] = mf
        mn = silu(jnp.dot(mf, wo1[...], preferred_element_type=jnp.float32)
                  + bo1[...])
        mn = mn + silu(jnp.dot(mf, wo2[...], preferred_element_type=jnp.float32)
                       + bo2[...])
        mn = mn + silu(jnp.dot(mf, wo3[...], preferred_element_type=jnp.float32)
                       + bo3[...])
        mout_ref[...] = m_ref[...] + mn

    edge_spec = pl.BlockSpec((BLK, HID), lambda i: (i, 0))
    rb_spec = pl.BlockSpec((BLK, NRAD), lambda i: (i, 0))
    sb_spec = pl.BlockSpec((BLK, 8), lambda i: (i, 0))

    def full(shape):
        return pl.BlockSpec(shape, lambda i: tuple(0 for _ in shape))

    return pl.pallas_call(
        body,
        grid=(E // BLK,),
        in_specs=[
            edge_spec, edge_spec, rb_spec, sb_spec,
            full((NRAD, HID)), full((1, HID)), full((HID, HID)), full((1, HID)),
            full((1, 8)),
            full((HID, HID)), full((1, HID)),
            full((HID, HID)), full((1, HID)),
            full((HID, HID)), full((1, HID)),
        ],
        out_specs=[edge_spec, edge_spec],
        out_shape=[jax.ShapeDtypeStruct((E, HID), jnp.float32)] * 2,
    )(m, mkj, rbkj, sbji, W_r1, b_r1, W_r2, b_r2, w8,
      W_o1, b_o1, W_o2, b_o2, W_o3, b_o3)


def _tc_nodes(h, agg2, Wu1h, Wu1a, b_u1, W_u2, b_u2):
    BLK = 1000

    def body(h_ref, agg_ref, wa, wb, bu1, wu2, bu2, hout_ref):
        hh = h_ref[...]
        agg = agg_ref[0] + agg_ref[1]
        t = jax.nn.silu(
            jnp.dot(hh, wa[...], preferred_element_type=jnp.float32)
            + jnp.dot(agg, wb[...], preferred_element_type=jnp.float32)
            + bu1[...])
        hout_ref[...] = hh + jnp.dot(t, wu2[...],
                                     preferred_element_type=jnp.float32) + bu2[...]

    def full(shape):
        return pl.BlockSpec(shape, lambda i: tuple(0 for _ in shape))

    return pl.pallas_call(
        body,
        grid=(N // BLK,),
        in_specs=[
            pl.BlockSpec((BLK, HID), lambda i: (i, 0)),
            pl.BlockSpec((NC, BLK, HID), lambda i: (0, i, 0)),
            full((HID, HID)), full((HID, HID)), full((1, HID)),
            full((HID, HID)), full((1, HID)),
        ],
        out_specs=pl.BlockSpec((BLK, HID), lambda i: (i, 0)),
        out_shape=jax.ShapeDtypeStruct((N, HID), jnp.float32),
    )(h, agg2, Wu1h, Wu1a, b_u1, W_u2, b_u2)


def kernel(h, m, radial_basis, spherical_basis, edge_index, triplets,
           W_r1, b_r1, W_r2, b_r2, W_sp, b_sp, W_u1, b_u1, W_u2, b_u2,
           W_o1, b_o1, W_o2, b_o2, W_o3, b_o3):
    idx_ji = triplets[:, 0].astype(jnp.int32).reshape(NCH, CHUNK)
    idx_kj = triplets[:, 1].astype(jnp.int32).reshape(NCH, CHUNK)
    dst2d = edge_index[1].astype(jnp.int32).reshape(NCH, CHUNK)

    sb8 = jnp.concatenate(
        [spherical_basis, jnp.ones((E, 1), jnp.float32)], axis=1)
    w8 = jnp.concatenate(
        [jnp.mean(W_sp, axis=1), jnp.mean(b_sp)[None]]).reshape(1, 8)

    mkj, rbkj, sbji = _sc_gather(m, radial_basis, sb8, idx_kj, idx_ji)
    mf, m_out = _tc_edges(
        m, mkj, rbkj, sbji, W_r1, b_r1.reshape(1, HID), W_r2,
        b_r2.reshape(1, HID), w8, W_o1, b_o1.reshape(1, HID),
        W_o2, b_o2.reshape(1, HID), W_o3, b_o3.reshape(1, HID))
    agg2 = _sc_scatter(mf, dst2d)
    h_out = _tc_nodes(h, agg2, W_u1[:HID], W_u1[HID:], b_u1.reshape(1, HID),
                      W_u2, b_u2.reshape(1, HID))
    return (h_out, m_out)
